# baseline (speedup 1.0000x reference)
import functools

import jax
import jax.numpy as jnp
from jax import lax
from jax.experimental import pallas as pl
from jax.experimental.pallas import tpu as pltpu
from jax.experimental.pallas import tpu_sc as plsc

_NC = 2
_NS = 16
_NW = _NC * _NS
_K = 8
_R = 248


@functools.lru_cache(maxsize=None)
def _build_widen(v: int, d: int):
    np_full = v // 128
    tail = v - np_full * 128
    per_w = np_full // _NW
    extra = np_full - per_w * _NW
    assert per_w % 2 == 0 and tail % 8 == 0
    mesh = plsc.VectorSubcoreMesh(core_axis_name="c", subcore_axis_name="s")

    @functools.partial(
        pl.kernel,
        mesh=mesh,
        out_type=jax.ShapeDtypeStruct((v, 128), jnp.float32),
        scratch_types=[
            pltpu.VMEM((d, 128), jnp.float32),
            pltpu.VMEM((d, 128), jnp.float32),
            pltpu.VMEM((128, 128), jnp.float32),
            pltpu.VMEM((128, 128), jnp.float32),
            pltpu.VMEM((max(tail, 8), d), jnp.float32),
            pltpu.SemaphoreType.DMA,
            pltpu.SemaphoreType.DMA,
            pltpu.SemaphoreType.DMA,
            pltpu.SemaphoreType.DMA,
        ],
        compiler_params=pltpu.CompilerParams(use_tc_tiling_on_sc=True,
                                             needs_layout_passes=False),
    )
    def widen(tablet_hbm, tail_hbm, wide_hbm, pbuf_a, pbuf_b, wbuf_a,
              wbuf_b, tbuf, isem_a, isem_b, osem_a, osem_b):
        wid = lax.axis_index("s") * _NC + lax.axis_index("c")
        start = wid * per_w + jnp.minimum(wid, extra)
        count = per_w + jnp.where(wid < extra, 1, 0)
        iota = lax.iota(jnp.int32, 16)

        def fire_in(p, buf, sem):
            pltpu.async_copy(tablet_hbm.at[:, pl.ds(p * 128, 128)], buf, sem)

        def wait_in(p, buf, sem):
            pltpu.make_async_copy(
                tablet_hbm.at[:, pl.ds(p * 128, 128)], buf, sem).wait()

        def fire_out(p, buf, sem):
            pltpu.async_copy(buf, wide_hbm.at[pl.ds(p * 128, 128)], sem)

        def wait_out(p, buf, sem):
            pltpu.make_async_copy(
                buf, wide_hbm.at[pl.ds(p * 128, 128)], sem).wait()

        def transpose(pbuf, wbuf):
            rows = [iota + 16 * m for m in range(d // 16)]
            for vv in range(128):
                col = jnp.full((16,), vv, jnp.int32)
                for m in range(d // 16):
                    x = plsc.load_gather(pbuf, [rows[m], col])
                    wbuf[vv, pl.ds(16 * m, 16)] = x

        fire_in(start, pbuf_a, isem_a)

        def body(i, carry):
            pa = start + 2 * i
            pb = pa + 1
            fire_in(pb, pbuf_b, isem_b)
            wait_in(pa, pbuf_a, isem_a)

            @pl.when(i > 0)
            def _():
                wait_out(pa - 2, wbuf_a, osem_a)

            transpose(pbuf_a, wbuf_a)
            fire_out(pa, wbuf_a, osem_a)

            @pl.when(i < per_w // 2 - 1)
            def _():
                fire_in(pa + 2, pbuf_a, isem_a)

            wait_in(pb, pbuf_b, isem_b)

            @pl.when(i > 0)
            def _():
                wait_out(pb - 2, wbuf_b, osem_b)

            transpose(pbuf_b, wbuf_b)
            fire_out(pb, wbuf_b, osem_b)
            return carry

        lax.fori_loop(0, per_w // 2, body, 0)
        wait_out(start + per_w - 2, wbuf_a, osem_a)
        wait_out(start + per_w - 1, wbuf_b, osem_b)

        @pl.when(wid < extra)
        def _():
            p = start + per_w
            fire_in(p, pbuf_a, isem_a)
            wait_in(p, pbuf_a, isem_a)
            transpose(pbuf_a, wbuf_a)
            fire_out(p, wbuf_a, osem_a)
            wait_out(p, wbuf_a, osem_a)

        if tail:
            @pl.when(wid == _NW - 1)
            def _():
                pltpu.sync_copy(tail_hbm, tbuf)

                def rows(i, carry):
                    for k in range(8):
                        r = i * 8 + k
                        for c in range(0, d, 16):
                            wbuf_a[r, pl.ds(c, 16)] = tbuf[r, pl.ds(c, 16)]
                    return carry
                lax.fori_loop(0, tail // 8, rows, 0)
                pltpu.sync_copy(wbuf_a.at[pl.ds(0, tail)],
                                wide_hbm.at[pl.ds(np_full * 128, tail)])

    return widen


@functools.lru_cache(maxsize=None)
def _build_gather(b: int, l: int, v: int):
    bw = b // _NW
    nph = bw // _K
    assert bw * _NW == b and nph * _K == bw and nph % 2 == 0
    mesh = plsc.VectorSubcoreMesh(core_axis_name="c", subcore_axis_name="s")

    @functools.partial(
        pl.kernel,
        mesh=mesh,
        out_type=jax.ShapeDtypeStruct((b, l, 128), jnp.float32),
        scratch_types=[
            pltpu.VMEM((bw, l), jnp.int32),
            pltpu.VMEM((_K, l, 128), jnp.float32),
            pltpu.VMEM((_K, l, 128), jnp.float32),
            pltpu.SemaphoreType.DMA,
            pltpu.SemaphoreType.DMA,
            pltpu.SemaphoreType.DMA,
            pltpu.SemaphoreType.DMA,
        ],
        compiler_params=pltpu.CompilerParams(use_tc_tiling_on_sc=True),
    )
    def gather(wide_hbm, idx_hbm, out_hbm, idx_v, buf_a, buf_b,
               gsem_a, gsem_b, ssem_a, ssem_b):
        wid = lax.axis_index("s") * _NC + lax.axis_index("c")
        base = wid * bw
        pltpu.sync_copy(idx_hbm.at[pl.ds(base, bw)], idx_v)

        def fire_gathers(phase, buf, sem):
            for i in range(_K):
                pltpu.async_copy(
                    wide_hbm.at[idx_v.at[phase * _K + i]], buf.at[i], sem)

        def drain_gathers(phase, buf, sem):
            for i in range(_K):
                pltpu.make_async_copy(
                    wide_hbm.at[idx_v.at[phase * _K + i]], buf.at[i], sem
                ).wait()

        def fire_scatter(phase, buf, sem):
            pltpu.async_copy(
                buf, out_hbm.at[pl.ds(base + phase * _K, _K)], sem)

        def drain_scatter(phase, buf, sem):
            pltpu.make_async_copy(
                buf, out_hbm.at[pl.ds(base + phase * _K, _K)], sem).wait()

        fire_gathers(0, buf_a, gsem_a)

        def body(i, carry):
            pa = 2 * i
            pb = 2 * i + 1

            @pl.when(i > 0)
            def _():
                drain_scatter(pb - 2, buf_b, ssem_b)

            fire_gathers(pb, buf_b, gsem_b)
            drain_gathers(pa, buf_a, gsem_a)
            fire_scatter(pa, buf_a, ssem_a)

            @pl.when(i < nph // 2 - 1)
            def _():
                drain_scatter(pa, buf_a, ssem_a)
                fire_gathers(pa + 2, buf_a, gsem_a)

            drain_gathers(pb, buf_b, gsem_b)
            fire_scatter(pb, buf_b, ssem_b)
            return carry

        lax.fori_loop(0, nph // 2, body, 0)
        drain_scatter(nph - 2, buf_a, ssem_a)
        drain_scatter(nph - 1, buf_b, ssem_b)

    return gather


def kernel(indices, table):
    b, l = indices.shape
    v, d = table.shape
    np_full = v // 128
    tail = v - np_full * 128
    tablet = jnp.swapaxes(table, 0, 1)
    tail_arr = table[v - max(tail, 8):, :]
    wide = _build_widen(v, d)(tablet, tail_arr)
    gout = _build_gather(b, l, v)(wide, indices)
    return gout[:, :, :d], jnp.full((b,), l, dtype=jnp.int64)


# speedup vs baseline: 2.7684x; 2.7684x over previous
import functools

import jax
import jax.numpy as jnp
from jax import lax
from jax.experimental import pallas as pl
from jax.experimental.pallas import tpu as pltpu
from jax.experimental.pallas import tpu_sc as plsc

_NC = 2
_NS = 16
_NW = _NC * _NS
_K = 8
_R = 248


@functools.lru_cache(maxsize=None)
def _build_widen(v: int, d: int):
    np_full = v // 128
    tail = v - np_full * 128
    per_w = np_full // _NW
    extra = np_full - per_w * _NW
    assert per_w % 2 == 0 and tail % 8 == 0
    mesh = plsc.VectorSubcoreMesh(core_axis_name="c", subcore_axis_name="s")

    @functools.partial(
        pl.kernel,
        mesh=mesh,
        out_type=jax.ShapeDtypeStruct((v, 128), jnp.float32),
        scratch_types=[
            pltpu.VMEM((d, 128), jnp.float32),
            pltpu.VMEM((d, 128), jnp.float32),
            pltpu.VMEM((128, 128), jnp.float32),
            pltpu.VMEM((128, 128), jnp.float32),
            pltpu.VMEM((max(tail, 8), d), jnp.float32),
            pltpu.SemaphoreType.DMA,
            pltpu.SemaphoreType.DMA,
            pltpu.SemaphoreType.DMA,
            pltpu.SemaphoreType.DMA,
        ],
        compiler_params=pltpu.CompilerParams(use_tc_tiling_on_sc=True,
                                             needs_layout_passes=False),
    )
    def widen(tablet_hbm, tail_hbm, wide_hbm, pbuf_a, pbuf_b, wbuf_a,
              wbuf_b, tbuf, isem_a, isem_b, osem_a, osem_b):
        wid = lax.axis_index("s") * _NC + lax.axis_index("c")
        start = wid * per_w + jnp.minimum(wid, extra)
        count = per_w + jnp.where(wid < extra, 1, 0)
        iota = lax.iota(jnp.int32, 16)

        def fire_in(p, buf, sem):
            pltpu.async_copy(tablet_hbm.at[:, pl.ds(p * 128, 128)], buf, sem)

        def wait_in(p, buf, sem):
            pltpu.make_async_copy(
                tablet_hbm.at[:, pl.ds(p * 128, 128)], buf, sem).wait()

        def fire_out(p, buf, sem):
            pltpu.async_copy(buf, wide_hbm.at[pl.ds(p * 128, 128)], sem)

        def wait_out(p, buf, sem):
            pltpu.make_async_copy(
                buf, wide_hbm.at[pl.ds(p * 128, 128)], sem).wait()

        def transpose(pbuf, wbuf):
            def sloop(s, carry):
                for m in range(d // 16):
                    rowm = iota + 16 * m
                    v = jnp.bitwise_and(iota + (s + 16 * m), 127)
                    x = plsc.load_gather(pbuf, [rowm, v])
                    plsc.store_scatter(wbuf, [v, rowm], x)
                return carry
            lax.fori_loop(0, 128, sloop, 0)

        fire_in(start, pbuf_a, isem_a)

        def body(i, carry):
            pa = start + 2 * i
            pb = pa + 1
            fire_in(pb, pbuf_b, isem_b)
            wait_in(pa, pbuf_a, isem_a)

            @pl.when(i > 0)
            def _():
                wait_out(pa - 2, wbuf_a, osem_a)

            transpose(pbuf_a, wbuf_a)
            fire_out(pa, wbuf_a, osem_a)

            @pl.when(i < per_w // 2 - 1)
            def _():
                fire_in(pa + 2, pbuf_a, isem_a)

            wait_in(pb, pbuf_b, isem_b)

            @pl.when(i > 0)
            def _():
                wait_out(pb - 2, wbuf_b, osem_b)

            transpose(pbuf_b, wbuf_b)
            fire_out(pb, wbuf_b, osem_b)
            return carry

        lax.fori_loop(0, per_w // 2, body, 0)
        wait_out(start + per_w - 2, wbuf_a, osem_a)
        wait_out(start + per_w - 1, wbuf_b, osem_b)

        @pl.when(wid < extra)
        def _():
            p = start + per_w
            fire_in(p, pbuf_a, isem_a)
            wait_in(p, pbuf_a, isem_a)
            transpose(pbuf_a, wbuf_a)
            fire_out(p, wbuf_a, osem_a)
            wait_out(p, wbuf_a, osem_a)

        if tail:
            @pl.when(wid == _NW - 1)
            def _():
                pltpu.sync_copy(tail_hbm, tbuf)

                def rows(i, carry):
                    for k in range(8):
                        r = i * 8 + k
                        for c in range(0, d, 16):
                            wbuf_a[r, pl.ds(c, 16)] = tbuf[r, pl.ds(c, 16)]
                    return carry
                lax.fori_loop(0, tail // 8, rows, 0)
                pltpu.sync_copy(wbuf_a.at[pl.ds(0, tail)],
                                wide_hbm.at[pl.ds(np_full * 128, tail)])

    return widen


@functools.lru_cache(maxsize=None)
def _build_gather(b: int, l: int, v: int):
    bw = b // _NW
    nph = bw // _K
    assert bw * _NW == b and nph * _K == bw and nph % 2 == 0
    mesh = plsc.VectorSubcoreMesh(core_axis_name="c", subcore_axis_name="s")

    @functools.partial(
        pl.kernel,
        mesh=mesh,
        out_type=jax.ShapeDtypeStruct((b, l, 128), jnp.float32),
        scratch_types=[
            pltpu.VMEM((bw, l), jnp.int32),
            pltpu.VMEM((_K, l, 128), jnp.float32),
            pltpu.VMEM((_K, l, 128), jnp.float32),
            pltpu.SemaphoreType.DMA,
            pltpu.SemaphoreType.DMA,
            pltpu.SemaphoreType.DMA,
            pltpu.SemaphoreType.DMA,
        ],
        compiler_params=pltpu.CompilerParams(use_tc_tiling_on_sc=True),
    )
    def gather(wide_hbm, idx_hbm, out_hbm, idx_v, buf_a, buf_b,
               gsem_a, gsem_b, ssem_a, ssem_b):
        wid = lax.axis_index("s") * _NC + lax.axis_index("c")
        base = wid * bw
        pltpu.sync_copy(idx_hbm.at[pl.ds(base, bw)], idx_v)

        def fire_gathers(phase, buf, sem):
            for i in range(_K):
                pltpu.async_copy(
                    wide_hbm.at[idx_v.at[phase * _K + i]], buf.at[i], sem)

        def drain_gathers(phase, buf, sem):
            for i in range(_K):
                pltpu.make_async_copy(
                    wide_hbm.at[idx_v.at[phase * _K + i]], buf.at[i], sem
                ).wait()

        def fire_scatter(phase, buf, sem):
            pltpu.async_copy(
                buf, out_hbm.at[pl.ds(base + phase * _K, _K)], sem)

        def drain_scatter(phase, buf, sem):
            pltpu.make_async_copy(
                buf, out_hbm.at[pl.ds(base + phase * _K, _K)], sem).wait()

        fire_gathers(0, buf_a, gsem_a)

        def body(i, carry):
            pa = 2 * i
            pb = 2 * i + 1

            @pl.when(i > 0)
            def _():
                drain_scatter(pb - 2, buf_b, ssem_b)

            fire_gathers(pb, buf_b, gsem_b)
            drain_gathers(pa, buf_a, gsem_a)
            fire_scatter(pa, buf_a, ssem_a)

            @pl.when(i < nph // 2 - 1)
            def _():
                drain_scatter(pa, buf_a, ssem_a)
                fire_gathers(pa + 2, buf_a, gsem_a)

            drain_gathers(pb, buf_b, gsem_b)
            fire_scatter(pb, buf_b, ssem_b)
            return carry

        lax.fori_loop(0, nph // 2, body, 0)
        drain_scatter(nph - 2, buf_a, ssem_a)
        drain_scatter(nph - 1, buf_b, ssem_b)

    return gather


def kernel(indices, table):
    b, l = indices.shape
    v, d = table.shape
    np_full = v // 128
    tail = v - np_full * 128
    tablet = jnp.swapaxes(table, 0, 1)
    tail_arr = table[v - max(tail, 8):, :]
    wide = _build_widen(v, d)(tablet, tail_arr)
    gout = _build_gather(b, l, v)(wide, indices)
    return gout[:, :, :d], jnp.full((b,), l, dtype=jnp.int64)


# speedup vs baseline: 2.8371x; 1.0248x over previous
import functools

import jax
import jax.numpy as jnp
from jax import lax
from jax.experimental import pallas as pl
from jax.experimental.pallas import tpu as pltpu
from jax.experimental.pallas import tpu_sc as plsc

_NC = 2
_NS = 16
_NW = _NC * _NS
_K = 8
_R = 248


@functools.lru_cache(maxsize=None)
def _build_widen(v: int, d: int):
    np_full = v // 128
    tail = v - np_full * 128
    per_w = np_full // _NW
    extra = np_full - per_w * _NW
    assert per_w % 2 == 0 and tail % 8 == 0
    mesh = plsc.VectorSubcoreMesh(core_axis_name="c", subcore_axis_name="s")

    @functools.partial(
        pl.kernel,
        mesh=mesh,
        out_type=jax.ShapeDtypeStruct((v, 128), jnp.float32),
        scratch_types=[
            pltpu.VMEM((d, 128), jnp.float32),
            pltpu.VMEM((d, 128), jnp.float32),
            pltpu.VMEM((128, 128), jnp.float32),
            pltpu.VMEM((128, 128), jnp.float32),
            pltpu.VMEM((max(tail, 8), d), jnp.float32),
            pltpu.SemaphoreType.DMA,
            pltpu.SemaphoreType.DMA,
            pltpu.SemaphoreType.DMA,
            pltpu.SemaphoreType.DMA,
        ],
        compiler_params=pltpu.CompilerParams(use_tc_tiling_on_sc=True,
                                             needs_layout_passes=False),
    )
    def widen(tablet_hbm, tail_hbm, wide_hbm, pbuf_a, pbuf_b, wbuf_a,
              wbuf_b, tbuf, isem_a, isem_b, osem_a, osem_b):
        wid = lax.axis_index("s") * _NC + lax.axis_index("c")
        start = wid * per_w + jnp.minimum(wid, extra)
        count = per_w + jnp.where(wid < extra, 1, 0)
        iota = lax.iota(jnp.int32, 16)

        def fire_in(p, buf, sem):
            pltpu.async_copy(tablet_hbm.at[:, pl.ds(p * 128, 128)], buf, sem)

        def wait_in(p, buf, sem):
            pltpu.make_async_copy(
                tablet_hbm.at[:, pl.ds(p * 128, 128)], buf, sem).wait()

        def fire_out(p, buf, sem):
            pltpu.async_copy(buf, wide_hbm.at[pl.ds(p * 128, 128)], sem)

        def wait_out(p, buf, sem):
            pltpu.make_async_copy(
                buf, wide_hbm.at[pl.ds(p * 128, 128)], sem).wait()

        def transpose(pbuf, wbuf):
            def sloop(s4, carry):
                s = s4 * 4
                for u in range(4):
                    for m in range(d // 16):
                        rowm = iota + 16 * m
                        v = jnp.bitwise_and(iota + (s + u + 16 * m), 127)
                        x = plsc.load_gather(pbuf, [rowm, v])
                        plsc.store_scatter(wbuf, [v, rowm], x)
                return carry
            lax.fori_loop(0, 32, sloop, 0)

        fire_in(start, pbuf_a, isem_a)

        def body(i, carry):
            pa = start + 2 * i
            pb = pa + 1
            fire_in(pb, pbuf_b, isem_b)
            wait_in(pa, pbuf_a, isem_a)

            @pl.when(i > 0)
            def _():
                wait_out(pa - 2, wbuf_a, osem_a)

            transpose(pbuf_a, wbuf_a)
            fire_out(pa, wbuf_a, osem_a)

            @pl.when(i < per_w // 2 - 1)
            def _():
                fire_in(pa + 2, pbuf_a, isem_a)

            wait_in(pb, pbuf_b, isem_b)

            @pl.when(i > 0)
            def _():
                wait_out(pb - 2, wbuf_b, osem_b)

            transpose(pbuf_b, wbuf_b)
            fire_out(pb, wbuf_b, osem_b)
            return carry

        lax.fori_loop(0, per_w // 2, body, 0)
        wait_out(start + per_w - 2, wbuf_a, osem_a)
        wait_out(start + per_w - 1, wbuf_b, osem_b)

        @pl.when(wid < extra)
        def _():
            p = start + per_w
            fire_in(p, pbuf_a, isem_a)
            wait_in(p, pbuf_a, isem_a)
            transpose(pbuf_a, wbuf_a)
            fire_out(p, wbuf_a, osem_a)
            wait_out(p, wbuf_a, osem_a)

        if tail:
            @pl.when(wid == _NW - 1)
            def _():
                pltpu.sync_copy(tail_hbm, tbuf)

                def rows(i, carry):
                    for k in range(8):
                        r = i * 8 + k
                        for c in range(0, d, 16):
                            wbuf_a[r, pl.ds(c, 16)] = tbuf[r, pl.ds(c, 16)]
                    return carry
                lax.fori_loop(0, tail // 8, rows, 0)
                pltpu.sync_copy(wbuf_a.at[pl.ds(0, tail)],
                                wide_hbm.at[pl.ds(np_full * 128, tail)])

    return widen


@functools.lru_cache(maxsize=None)
def _build_gather(b: int, l: int, v: int):
    bw = b // _NW
    nph = bw // _K
    assert bw * _NW == b and nph * _K == bw and nph % 2 == 0
    mesh = plsc.VectorSubcoreMesh(core_axis_name="c", subcore_axis_name="s")

    @functools.partial(
        pl.kernel,
        mesh=mesh,
        out_type=jax.ShapeDtypeStruct((b, l, 128), jnp.float32),
        scratch_types=[
            pltpu.VMEM((bw, l), jnp.int32),
            pltpu.VMEM((_K, l, 128), jnp.float32),
            pltpu.VMEM((_K, l, 128), jnp.float32),
            pltpu.SemaphoreType.DMA,
            pltpu.SemaphoreType.DMA,
            pltpu.SemaphoreType.DMA,
            pltpu.SemaphoreType.DMA,
        ],
        compiler_params=pltpu.CompilerParams(use_tc_tiling_on_sc=True),
    )
    def gather(wide_hbm, idx_hbm, out_hbm, idx_v, buf_a, buf_b,
               gsem_a, gsem_b, ssem_a, ssem_b):
        wid = lax.axis_index("s") * _NC + lax.axis_index("c")
        base = wid * bw
        pltpu.sync_copy(idx_hbm.at[pl.ds(base, bw)], idx_v)

        def fire_gathers(phase, buf, sem):
            for i in range(_K):
                pltpu.async_copy(
                    wide_hbm.at[idx_v.at[phase * _K + i]], buf.at[i], sem)

        def drain_gathers(phase, buf, sem):
            for i in range(_K):
                pltpu.make_async_copy(
                    wide_hbm.at[idx_v.at[phase * _K + i]], buf.at[i], sem
                ).wait()

        def fire_scatter(phase, buf, sem):
            pltpu.async_copy(
                buf, out_hbm.at[pl.ds(base + phase * _K, _K)], sem)

        def drain_scatter(phase, buf, sem):
            pltpu.make_async_copy(
                buf, out_hbm.at[pl.ds(base + phase * _K, _K)], sem).wait()

        fire_gathers(0, buf_a, gsem_a)

        def body(i, carry):
            pa = 2 * i
            pb = 2 * i + 1

            @pl.when(i > 0)
            def _():
                drain_scatter(pb - 2, buf_b, ssem_b)

            fire_gathers(pb, buf_b, gsem_b)
            drain_gathers(pa, buf_a, gsem_a)
            fire_scatter(pa, buf_a, ssem_a)

            @pl.when(i < nph // 2 - 1)
            def _():
                drain_scatter(pa, buf_a, ssem_a)
                fire_gathers(pa + 2, buf_a, gsem_a)

            drain_gathers(pb, buf_b, gsem_b)
            fire_scatter(pb, buf_b, ssem_b)
            return carry

        lax.fori_loop(0, nph // 2, body, 0)
        drain_scatter(nph - 2, buf_a, ssem_a)
        drain_scatter(nph - 1, buf_b, ssem_b)

    return gather


def kernel(indices, table):
    b, l = indices.shape
    v, d = table.shape
    np_full = v // 128
    tail = v - np_full * 128
    tablet = jnp.swapaxes(table, 0, 1)
    tail_arr = table[v - max(tail, 8):, :]
    wide = _build_widen(v, d)(tablet, tail_arr)
    gout = _build_gather(b, l, v)(wide, indices)
    return gout[:, :, :d], jnp.full((b,), l, dtype=jnp.int64)


# speedup vs baseline: 2.8391x; 1.0007x over previous
import functools

import jax
import jax.numpy as jnp
from jax import lax
from jax.experimental import pallas as pl
from jax.experimental.pallas import tpu as pltpu
from jax.experimental.pallas import tpu_sc as plsc

_NC = 2
_NS = 16
_NW = _NC * _NS
_K = 8
_R = 248


@functools.lru_cache(maxsize=None)
def _build_widen(v: int, d: int):
    np_full = v // 128
    tail = v - np_full * 128
    per_w = np_full // _NW
    extra = np_full - per_w * _NW
    assert per_w % 2 == 0 and tail % 8 == 0
    mesh = plsc.VectorSubcoreMesh(core_axis_name="c", subcore_axis_name="s")

    @functools.partial(
        pl.kernel,
        mesh=mesh,
        out_type=jax.ShapeDtypeStruct((v, 128), jnp.float32),
        scratch_types=[
            pltpu.VMEM((d, 128), jnp.float32),
            pltpu.VMEM((d, 128), jnp.float32),
            pltpu.VMEM((128, 128), jnp.float32),
            pltpu.VMEM((128, 128), jnp.float32),
            pltpu.VMEM((max(tail, 8), d), jnp.float32),
            pltpu.SemaphoreType.DMA,
            pltpu.SemaphoreType.DMA,
            pltpu.SemaphoreType.DMA,
            pltpu.SemaphoreType.DMA,
        ],
        compiler_params=pltpu.CompilerParams(use_tc_tiling_on_sc=True,
                                             needs_layout_passes=False),
    )
    def widen(tablet_hbm, tail_hbm, wide_hbm, pbuf_a, pbuf_b, wbuf_a,
              wbuf_b, tbuf, isem_a, isem_b, osem_a, osem_b):
        wid = lax.axis_index("s") * _NC + lax.axis_index("c")
        start = wid * per_w + jnp.minimum(wid, extra)
        count = per_w + jnp.where(wid < extra, 1, 0)
        iota = lax.iota(jnp.int32, 16)

        def fire_in(p, buf, sem):
            pltpu.async_copy(tablet_hbm.at[:, pl.ds(p * 128, 128)], buf, sem)

        def wait_in(p, buf, sem):
            pltpu.make_async_copy(
                tablet_hbm.at[:, pl.ds(p * 128, 128)], buf, sem).wait()

        def fire_out(p, buf, sem):
            pltpu.async_copy(buf, wide_hbm.at[pl.ds(p * 128, 128)], sem)

        def wait_out(p, buf, sem):
            pltpu.make_async_copy(
                buf, wide_hbm.at[pl.ds(p * 128, 128)], sem).wait()

        def transpose(pbuf, wbuf):
            def sloop(s4, carry):
                s = s4 * 4
                for u in range(4):
                    v = jnp.bitwise_and(iota + (s + u), 127)
                    for m in range(d // 16):
                        rowm = iota + 16 * m
                        x = plsc.load_gather(pbuf, [rowm, v])
                        plsc.store_scatter(wbuf, [v, rowm], x)
                return carry
            lax.fori_loop(0, 32, sloop, 0)

        fire_in(start, pbuf_a, isem_a)

        def body(i, carry):
            pa = start + 2 * i
            pb = pa + 1
            fire_in(pb, pbuf_b, isem_b)
            wait_in(pa, pbuf_a, isem_a)

            @pl.when(i > 0)
            def _():
                wait_out(pa - 2, wbuf_a, osem_a)

            transpose(pbuf_a, wbuf_a)
            fire_out(pa, wbuf_a, osem_a)

            @pl.when(i < per_w // 2 - 1)
            def _():
                fire_in(pa + 2, pbuf_a, isem_a)

            wait_in(pb, pbuf_b, isem_b)

            @pl.when(i > 0)
            def _():
                wait_out(pb - 2, wbuf_b, osem_b)

            transpose(pbuf_b, wbuf_b)
            fire_out(pb, wbuf_b, osem_b)
            return carry

        lax.fori_loop(0, per_w // 2, body, 0)
        wait_out(start + per_w - 2, wbuf_a, osem_a)
        wait_out(start + per_w - 1, wbuf_b, osem_b)

        @pl.when(wid < extra)
        def _():
            p = start + per_w
            fire_in(p, pbuf_a, isem_a)
            wait_in(p, pbuf_a, isem_a)
            transpose(pbuf_a, wbuf_a)
            fire_out(p, wbuf_a, osem_a)
            wait_out(p, wbuf_a, osem_a)

        if tail:
            @pl.when(wid == _NW - 1)
            def _():
                pltpu.sync_copy(tail_hbm, tbuf)

                def rows(i, carry):
                    for k in range(8):
                        r = i * 8 + k
                        for c in range(0, d, 16):
                            wbuf_a[r, pl.ds(c, 16)] = tbuf[r, pl.ds(c, 16)]
                    return carry
                lax.fori_loop(0, tail // 8, rows, 0)
                pltpu.sync_copy(wbuf_a.at[pl.ds(0, tail)],
                                wide_hbm.at[pl.ds(np_full * 128, tail)])

    return widen


@functools.lru_cache(maxsize=None)
def _build_gather(b: int, l: int, v: int):
    bw = b // _NW
    nph = bw // _K
    assert bw * _NW == b and nph * _K == bw and nph % 2 == 0
    mesh = plsc.VectorSubcoreMesh(core_axis_name="c", subcore_axis_name="s")

    @functools.partial(
        pl.kernel,
        mesh=mesh,
        out_type=jax.ShapeDtypeStruct((b, l, 128), jnp.float32),
        scratch_types=[
            pltpu.VMEM((bw, l), jnp.int32),
            pltpu.VMEM((_K, l, 128), jnp.float32),
            pltpu.VMEM((_K, l, 128), jnp.float32),
            pltpu.SemaphoreType.DMA,
            pltpu.SemaphoreType.DMA,
            pltpu.SemaphoreType.DMA,
            pltpu.SemaphoreType.DMA,
        ],
        compiler_params=pltpu.CompilerParams(use_tc_tiling_on_sc=True),
    )
    def gather(wide_hbm, idx_hbm, out_hbm, idx_v, buf_a, buf_b,
               gsem_a, gsem_b, ssem_a, ssem_b):
        wid = lax.axis_index("s") * _NC + lax.axis_index("c")
        base = wid * bw
        pltpu.sync_copy(idx_hbm.at[pl.ds(base, bw)], idx_v)

        def fire_gathers(phase, buf, sem):
            for i in range(_K):
                pltpu.async_copy(
                    wide_hbm.at[idx_v.at[phase * _K + i]], buf.at[i], sem)

        def drain_gathers(phase, buf, sem):
            for i in range(_K):
                pltpu.make_async_copy(
                    wide_hbm.at[idx_v.at[phase * _K + i]], buf.at[i], sem
                ).wait()

        def fire_scatter(phase, buf, sem):
            pltpu.async_copy(
                buf, out_hbm.at[pl.ds(base + phase * _K, _K)], sem)

        def drain_scatter(phase, buf, sem):
            pltpu.make_async_copy(
                buf, out_hbm.at[pl.ds(base + phase * _K, _K)], sem).wait()

        fire_gathers(0, buf_a, gsem_a)

        def body(i, carry):
            pa = 2 * i
            pb = 2 * i + 1

            @pl.when(i > 0)
            def _():
                drain_scatter(pb - 2, buf_b, ssem_b)

            fire_gathers(pb, buf_b, gsem_b)
            drain_gathers(pa, buf_a, gsem_a)
            fire_scatter(pa, buf_a, ssem_a)

            @pl.when(i < nph // 2 - 1)
            def _():
                drain_scatter(pa, buf_a, ssem_a)
                fire_gathers(pa + 2, buf_a, gsem_a)

            drain_gathers(pb, buf_b, gsem_b)
            fire_scatter(pb, buf_b, ssem_b)
            return carry

        lax.fori_loop(0, nph // 2, body, 0)
        drain_scatter(nph - 2, buf_a, ssem_a)
        drain_scatter(nph - 1, buf_b, ssem_b)

    return gather


def kernel(indices, table):
    b, l = indices.shape
    v, d = table.shape
    np_full = v // 128
    tail = v - np_full * 128
    tablet = jnp.swapaxes(table, 0, 1)
    tail_arr = table[v - max(tail, 8):, :]
    wide = _build_widen(v, d)(tablet, tail_arr)
    gout = _build_gather(b, l, v)(wide, indices)
    return gout[:, :, :d], jnp.full((b,), l, dtype=jnp.int64)


# speedup vs baseline: 4.2822x; 1.5083x over previous
import functools

import jax
import jax.numpy as jnp
from jax import lax
from jax.experimental import pallas as pl
from jax.experimental.pallas import tpu as pltpu
from jax.experimental.pallas import tpu_sc as plsc

_NC = 2
_NS = 16
_NW = _NC * _NS
_K = 8
_R = 248


@functools.lru_cache(maxsize=None)
def _build_widen(v: int, d: int):
    np_full = v // 128
    tail = v - np_full * 128
    per_w = np_full // _NW
    extra = np_full - per_w * _NW
    assert per_w % 2 == 0 and tail % 8 == 0
    mesh = plsc.VectorSubcoreMesh(core_axis_name="c", subcore_axis_name="s")

    @functools.partial(
        pl.kernel,
        mesh=mesh,
        out_type=jax.ShapeDtypeStruct((v, 128), jnp.float32),
        scratch_types=[
            pltpu.VMEM((d, 128), jnp.float32),
            pltpu.VMEM((d, 128), jnp.float32),
            pltpu.VMEM((128, 128), jnp.float32),
            pltpu.VMEM((128, 128), jnp.float32),
            pltpu.VMEM((max(tail, 8), d), jnp.float32),
            pltpu.SemaphoreType.DMA,
            pltpu.SemaphoreType.DMA,
            pltpu.SemaphoreType.DMA,
            pltpu.SemaphoreType.DMA,
        ],
        compiler_params=pltpu.CompilerParams(use_tc_tiling_on_sc=True,
                                             needs_layout_passes=False),
    )
    def widen(tablet_hbm, tail_hbm, wide_hbm, pbuf_a, pbuf_b, wbuf_a,
              wbuf_b, tbuf, isem_a, isem_b, osem_a, osem_b):
        wid = lax.axis_index("s") * _NC + lax.axis_index("c")
        start = wid * per_w + jnp.minimum(wid, extra)
        count = per_w + jnp.where(wid < extra, 1, 0)
        iota = lax.iota(jnp.int32, 16)

        def fire_in(p, buf, sem):
            pltpu.async_copy(tablet_hbm.at[:, pl.ds(p * 128, 128)], buf, sem)

        def wait_in(p, buf, sem):
            pltpu.make_async_copy(
                tablet_hbm.at[:, pl.ds(p * 128, 128)], buf, sem).wait()

        def fire_out(p, buf, sem):
            pltpu.async_copy(buf, wide_hbm.at[pl.ds(p * 128, 128)], sem)

        def wait_out(p, buf, sem):
            pltpu.make_async_copy(
                buf, wide_hbm.at[pl.ds(p * 128, 128)], sem).wait()

        def transpose(pbuf, wbuf):
            def sloop(s4, carry):
                s = s4 * 4
                vs = [jnp.bitwise_and(iota + (s + u), 127) for u in range(4)]
                xs = []
                for u in range(4):
                    for m in range(d // 16):
                        rowm = iota + 16 * m
                        xs.append(plsc.load_gather(pbuf, [rowm, vs[u]]))
                i = 0
                for u in range(4):
                    for m in range(d // 16):
                        rowm = iota + 16 * m
                        plsc.store_scatter(wbuf, [vs[u], rowm], xs[i])
                        i += 1
                return carry
            lax.fori_loop(0, 32, sloop, 0)

        fire_in(start, pbuf_a, isem_a)

        def body(i, carry):
            pa = start + 2 * i
            pb = pa + 1
            fire_in(pb, pbuf_b, isem_b)
            wait_in(pa, pbuf_a, isem_a)

            @pl.when(i > 0)
            def _():
                wait_out(pa - 2, wbuf_a, osem_a)

            transpose(pbuf_a, wbuf_a)
            fire_out(pa, wbuf_a, osem_a)

            @pl.when(i < per_w // 2 - 1)
            def _():
                fire_in(pa + 2, pbuf_a, isem_a)

            wait_in(pb, pbuf_b, isem_b)

            @pl.when(i > 0)
            def _():
                wait_out(pb - 2, wbuf_b, osem_b)

            transpose(pbuf_b, wbuf_b)
            fire_out(pb, wbuf_b, osem_b)
            return carry

        lax.fori_loop(0, per_w // 2, body, 0)
        wait_out(start + per_w - 2, wbuf_a, osem_a)
        wait_out(start + per_w - 1, wbuf_b, osem_b)

        @pl.when(wid < extra)
        def _():
            p = start + per_w
            fire_in(p, pbuf_a, isem_a)
            wait_in(p, pbuf_a, isem_a)
            transpose(pbuf_a, wbuf_a)
            fire_out(p, wbuf_a, osem_a)
            wait_out(p, wbuf_a, osem_a)

        if tail:
            @pl.when(wid == _NW - 1)
            def _():
                pltpu.sync_copy(tail_hbm, tbuf)

                def rows(i, carry):
                    for k in range(8):
                        r = i * 8 + k
                        for c in range(0, d, 16):
                            wbuf_a[r, pl.ds(c, 16)] = tbuf[r, pl.ds(c, 16)]
                    return carry
                lax.fori_loop(0, tail // 8, rows, 0)
                pltpu.sync_copy(wbuf_a.at[pl.ds(0, tail)],
                                wide_hbm.at[pl.ds(np_full * 128, tail)])

    return widen


@functools.lru_cache(maxsize=None)
def _build_gather(b: int, l: int, v: int):
    bw = b // _NW
    nph = bw // _K
    assert bw * _NW == b and nph * _K == bw and nph % 2 == 0
    mesh = plsc.VectorSubcoreMesh(core_axis_name="c", subcore_axis_name="s")

    @functools.partial(
        pl.kernel,
        mesh=mesh,
        out_type=jax.ShapeDtypeStruct((b, l, 128), jnp.float32),
        scratch_types=[
            pltpu.VMEM((bw, l), jnp.int32),
            pltpu.VMEM((_K, l, 128), jnp.float32),
            pltpu.VMEM((_K, l, 128), jnp.float32),
            pltpu.SemaphoreType.DMA,
            pltpu.SemaphoreType.DMA,
            pltpu.SemaphoreType.DMA,
            pltpu.SemaphoreType.DMA,
        ],
        compiler_params=pltpu.CompilerParams(use_tc_tiling_on_sc=True),
    )
    def gather(wide_hbm, idx_hbm, out_hbm, idx_v, buf_a, buf_b,
               gsem_a, gsem_b, ssem_a, ssem_b):
        wid = lax.axis_index("s") * _NC + lax.axis_index("c")
        base = wid * bw
        pltpu.sync_copy(idx_hbm.at[pl.ds(base, bw)], idx_v)

        def fire_gathers(phase, buf, sem):
            for i in range(_K):
                pltpu.async_copy(
                    wide_hbm.at[idx_v.at[phase * _K + i]], buf.at[i], sem)

        def drain_gathers(phase, buf, sem):
            for i in range(_K):
                pltpu.make_async_copy(
                    wide_hbm.at[idx_v.at[phase * _K + i]], buf.at[i], sem
                ).wait()

        def fire_scatter(phase, buf, sem):
            pltpu.async_copy(
                buf, out_hbm.at[pl.ds(base + phase * _K, _K)], sem)

        def drain_scatter(phase, buf, sem):
            pltpu.make_async_copy(
                buf, out_hbm.at[pl.ds(base + phase * _K, _K)], sem).wait()

        fire_gathers(0, buf_a, gsem_a)

        def body(i, carry):
            pa = 2 * i
            pb = 2 * i + 1

            @pl.when(i > 0)
            def _():
                drain_scatter(pb - 2, buf_b, ssem_b)

            fire_gathers(pb, buf_b, gsem_b)
            drain_gathers(pa, buf_a, gsem_a)
            fire_scatter(pa, buf_a, ssem_a)

            @pl.when(i < nph // 2 - 1)
            def _():
                drain_scatter(pa, buf_a, ssem_a)
                fire_gathers(pa + 2, buf_a, gsem_a)

            drain_gathers(pb, buf_b, gsem_b)
            fire_scatter(pb, buf_b, ssem_b)
            return carry

        lax.fori_loop(0, nph // 2, body, 0)
        drain_scatter(nph - 2, buf_a, ssem_a)
        drain_scatter(nph - 1, buf_b, ssem_b)

    return gather


def kernel(indices, table):
    b, l = indices.shape
    v, d = table.shape
    np_full = v // 128
    tail = v - np_full * 128
    tablet = jnp.swapaxes(table, 0, 1)
    tail_arr = table[v - max(tail, 8):, :]
    wide = _build_widen(v, d)(tablet, tail_arr)
    gout = _build_gather(b, l, v)(wide, indices)
    return gout[:, :, :d], jnp.full((b,), l, dtype=jnp.int64)
